# baseline (speedup 1.0000x reference)
import functools

import jax
import jax.numpy as jnp
from jax import lax
from jax.experimental import pallas as pl
from jax.experimental.pallas import tpu as pltpu
from jax.experimental.pallas import tpu_sc as plsc

NC = 2
NS = 16
NW = NC * NS
LANES = 16


def _mesh():
  return plsc.VectorSubcoreMesh(
      core_axis_name="c", subcore_axis_name="s", num_cores=NC, num_subcores=NS
  )


def _make_layer1(E, NPAD, D, NCHUNK, CH):
  EPC = E // NS
  HC = EPC // 5
  assert EPC % HC == 0 and HC % LANES == 0 and HC % 8 == 0
  ROWS = NPAD // NS

  @functools.partial(
      pl.kernel,
      out_type=(
          jax.ShapeDtypeStruct((NC, NPAD, D), jnp.float32),
          jax.ShapeDtypeStruct((NPAD,), jnp.float32),
          jax.ShapeDtypeStruct((NC, NPAD, D), jnp.float32),
      ),
      mesh=_mesh(),
      compiler_params=pltpu.CompilerParams(
          needs_layout_passes=False, use_tc_tiling_on_sc=False
      ),
      scratch_types=[
          pltpu.VMEM((HC,), jnp.int32),
          pltpu.VMEM((NPAD,), jnp.float32),
          pltpu.VMEM((ROWS,), jnp.float32),
          pltpu.VMEM((ROWS,), jnp.float32),
          pltpu.VMEM((ROWS,), jnp.float32),
          pltpu.VMEM((ROWS, D), jnp.float32),
          pltpu.VMEM((NCHUNK, CH), jnp.int32),
          pltpu.VMEM((NCHUNK, CH), jnp.int32),
          pltpu.VMEM((8, CH, D), jnp.float32),
          pltpu.VMEM_SHARED((NS, NPAD), jnp.float32),
          pltpu.VMEM_SHARED((NPAD, D), jnp.float32),
          pltpu.SemaphoreType.DMA,
          pltpu.SemaphoreType.DMA,
      ],
  )
  def layer1_kernel(col_hbm, rowr_hbm, colr_hbm, xw_hbm, zeros_hbm,
                    out_hbm, dis_hbm, y_hbm,
                    colh, hist, acc, tmp, disv, xwv, rowv, colv, buf,
                    shared, accp, gsem, ssem):
    cid = lax.axis_index("c")
    sid = lax.axis_index("s")
    wid = sid * NC + cid
    base = sid * ROWS

    zeros16 = jnp.zeros((LANES,), jnp.float32)

    def zero_body(i, _):
      hist[pl.ds(i * LANES, LANES)] = zeros16
      return ()

    lax.fori_loop(0, NPAD // LANES, zero_body, (), unroll=8)

    def chunk_body(c, _):
      pltpu.sync_copy(col_hbm.at[pl.ds(sid * EPC + c * HC, HC)], colh)

      def hist_body(i, _):
        idx = colh[pl.ds(i * LANES, LANES)]
        cnt, last = plsc.scan_count(idx)
        plsc.addupdate_scatter(hist, [idx], cnt.astype(jnp.float32),
                               mask=last)
        return ()

      lax.fori_loop(0, HC // LANES, hist_body, (), unroll=4)
      return ()

    lax.fori_loop(0, EPC // HC, chunk_body, ())

    pltpu.sync_copy(zeros_hbm.at[pl.ds(base, ROWS)], accp.at[pl.ds(base, ROWS)])
    pltpu.sync_copy(rowr_hbm.at[wid], rowv)
    pltpu.sync_copy(colr_hbm.at[wid], colv)

    pltpu.sync_copy(hist, shared.at[sid])
    plsc.subcore_barrier()

    pltpu.sync_copy(shared.at[0, pl.ds(base, ROWS)], acc)

    def red_body(k, _):
      pltpu.sync_copy(shared.at[k, pl.ds(base, ROWS)], tmp)

      def add_body(i, _):
        s = pl.ds(i * LANES, LANES)
        acc[s] = acc[s] + tmp[s]
        return ()

      lax.fori_loop(0, ROWS // LANES, add_body, (), unroll=8)
      return ()

    lax.fori_loop(1, NS, red_body, ())

    def dis_body(i, _):
      s = pl.ds(i * LANES, LANES)
      d = acc[s] + 1.0
      seed = plsc.bitcast(
          jnp.int32(0x5F3759DF) - (plsc.bitcast(d, jnp.int32) >> 1),
          jnp.float32)
      hd = 0.5 * d
      y = seed * (1.5 - hd * seed * seed)
      y = y * (1.5 - hd * y * y)
      y = y * (1.5 - hd * y * y)
      disv[s] = y
      return ()

    lax.fori_loop(0, ROWS // LANES, dis_body, (), unroll=4)

    @pl.when(cid == 0)
    def _():
      pltpu.sync_copy(disv, dis_hbm.at[pl.ds(base, ROWS)])

    pltpu.sync_copy(xw_hbm.at[pl.ds(base, ROWS)], xwv)

    def scale_body(i, _):
      dis16 = disv[pl.ds(i * LANES, LANES)]
      for k in range(LANES):
        r = i * LANES + k
        s = dis16[k]
        for h0 in range(D // LANES):
          sl = pl.ds(h0 * LANES, LANES)
          xwv[r, sl] = xwv[r, sl] * s
      return ()

    lax.fori_loop(0, ROWS // LANES, scale_body, ())
    pltpu.sync_copy(xwv, y_hbm.at[cid, pl.ds(base, ROWS)])
    plsc.subcore_barrier()

    ysrc = y_hbm.at[cid]
    NBUF = 8
    for p in range(NBUF - 1):
      pltpu.async_copy(ysrc.at[rowv.at[p]], buf.at[p], gsem)

    def step(j, _):
      slot = lax.rem(j, NBUF)
      pslot = lax.rem(j + NBUF - 1, NBUF)

      @pl.when(j >= 1)
      def _():
        pltpu.make_async_copy(buf.at[pslot], accp.at[colv.at[j - 1]],
                              ssem).wait()

      @pl.when(j + NBUF - 1 < NCHUNK)
      def _():
        pltpu.async_copy(ysrc.at[rowv.at[j + NBUF - 1]], buf.at[pslot], gsem)

      pltpu.make_async_copy(ysrc.at[rowv.at[j]], buf.at[slot], gsem).wait()
      pltpu.async_copy(buf.at[slot], accp.at[colv.at[j]], ssem, add=True)
      return ()

    lax.fori_loop(0, NCHUNK, step, ())
    last = NCHUNK - 1
    pltpu.make_async_copy(buf.at[lax.rem(last, NBUF)], accp.at[colv.at[last]],
                          ssem).wait()

    plsc.subcore_barrier()
    pltpu.sync_copy(accp.at[pl.ds(base, ROWS)],
                    out_hbm.at[cid, pl.ds(base, ROWS)])

  return layer1_kernel


def _make_prop(E, NPAD, D, NCHUNK, CH):
  ROWS = NPAD // NS

  @functools.partial(
      pl.kernel,
      out_type=jax.ShapeDtypeStruct((NC, NPAD, D), jnp.float32),
      mesh=_mesh(),
      compiler_params=pltpu.CompilerParams(
          needs_layout_passes=False, use_tc_tiling_on_sc=False
      ),
      scratch_types=[
          pltpu.VMEM((NCHUNK, CH), jnp.int32),
          pltpu.VMEM((NCHUNK, CH), jnp.int32),
          pltpu.VMEM((16, CH, D), jnp.float32),
          pltpu.VMEM_SHARED((NPAD, D), jnp.float32),
          pltpu.SemaphoreType.DMA,
          pltpu.SemaphoreType.DMA,
      ],
  )
  def prop_kernel(rowr_hbm, colr_hbm, y_hbm, zeros_hbm, out_hbm,
                  rowv, colv, buf, acc, gsem, ssem):
    cid = lax.axis_index("c")
    sid = lax.axis_index("s")
    wid = sid * NC + cid
    base = sid * ROWS

    pltpu.sync_copy(zeros_hbm.at[pl.ds(base, ROWS)], acc.at[pl.ds(base, ROWS)])
    pltpu.sync_copy(rowr_hbm.at[wid], rowv)
    pltpu.sync_copy(colr_hbm.at[wid], colv)
    plsc.subcore_barrier()

    NBUF = 16
    for p in range(NBUF - 1):
      pltpu.async_copy(y_hbm.at[rowv.at[p]], buf.at[p], gsem)

    def step(j, _):
      slot = lax.rem(j, NBUF)
      pslot = lax.rem(j + NBUF - 1, NBUF)

      @pl.when(j >= 1)
      def _():
        pltpu.make_async_copy(buf.at[pslot], acc.at[colv.at[j - 1]],
                              ssem).wait()

      @pl.when(j + NBUF - 1 < NCHUNK)
      def _():
        pltpu.async_copy(y_hbm.at[rowv.at[j + NBUF - 1]], buf.at[pslot], gsem)

      pltpu.make_async_copy(y_hbm.at[rowv.at[j]], buf.at[slot], gsem).wait()
      pltpu.async_copy(buf.at[slot], acc.at[colv.at[j]], ssem, add=True)
      return ()

    lax.fori_loop(0, NCHUNK, step, ())
    last = NCHUNK - 1
    pltpu.make_async_copy(buf.at[lax.rem(last, NBUF)], acc.at[colv.at[last]],
                          ssem).wait()

    plsc.subcore_barrier()
    pltpu.sync_copy(acc.at[pl.ds(base, ROWS)],
                    out_hbm.at[cid, pl.ds(base, ROWS)])

  return prop_kernel


def _mm_body(x_ref, w_ref, o_ref):
  o_ref[...] = jnp.dot(x_ref[...], w_ref[...],
                       preferred_element_type=jnp.float32)


def _mid_body(t_ref, y1_ref, dis_ref, w2_ref, b1_ref, y2_ref):
  dis = dis_ref[...]
  t = jnp.sum(t_ref[...], axis=0) + y1_ref[...]
  h = jnp.maximum(t * dis + b1_ref[...], 0.0)
  y2_ref[...] = jnp.dot(h, w2_ref[...], preferred_element_type=jnp.float32) * dis


def _final_body(t_ref, y2_ref, dis_ref, b2_ref, z_ref):
  z_ref[...] = ((jnp.sum(t_ref[...], axis=0) + y2_ref[...]) * dis_ref[...]
                + b2_ref[...])


def kernel(x, ei, W1, b1, W2, b2):
  N, F = x.shape
  E = ei.shape[1]
  H = W1.shape[1]
  EM = W2.shape[1]

  assert E % (NW * 8) == 0
  EPW = E // NW
  CH = next(c for c in range(128, 0, -1) if EPW % c == 0)
  NCHUNK = EPW // CH
  ROWS = -(-N // NS)
  ROWS = -(ROWS // -LANES) * LANES
  NPAD = NS * ROWS

  BR = 400
  assert N % BR == 0
  GN = N // BR

  rowr = ei[0].reshape(NW, NCHUNK, CH)
  colr = ei[1].reshape(NW, NCHUNK, CH)
  zeros_h = jnp.zeros((NPAD, H), jnp.float32)
  zeros_e = jnp.zeros((NPAD, EM), jnp.float32)

  layer1_fn = _make_layer1(E, NPAD, H, NCHUNK, CH)
  prop_e = _make_prop(E, NPAD, EM, NCHUNK, CH)

  xw1 = pl.pallas_call(
      _mm_body,
      grid=(GN,),
      in_specs=[
          pl.BlockSpec((BR, F), lambda i: (i, 0)),
          pl.BlockSpec((F, H), lambda i: (0, 0)),
      ],
      out_specs=pl.BlockSpec((BR, H), lambda i: (i, 0)),
      out_shape=jax.ShapeDtypeStruct((NPAD, H), jnp.float32),
  )(x, W1)

  T1, dis_flat, y1buf = layer1_fn(ei[1], rowr, colr, xw1, zeros_h)
  dis = dis_flat[:N].reshape(N, 1)
  y1 = y1buf[0, :N]

  y2 = pl.pallas_call(
      _mid_body,
      grid=(GN,),
      in_specs=[
          pl.BlockSpec((NC, BR, H), lambda i: (0, i, 0)),
          pl.BlockSpec((BR, H), lambda i: (i, 0)),
          pl.BlockSpec((BR, 1), lambda i: (i, 0)),
          pl.BlockSpec((H, EM), lambda i: (0, 0)),
          pl.BlockSpec((1, H), lambda i: (0, 0)),
      ],
      out_specs=pl.BlockSpec((BR, EM), lambda i: (i, 0)),
      out_shape=jax.ShapeDtypeStruct((N, EM), jnp.float32),
  )(T1, y1, dis, W2, b1.reshape(1, H))

  T2 = prop_e(rowr, colr, y2, zeros_e)

  z = pl.pallas_call(
      _final_body,
      grid=(GN,),
      in_specs=[
          pl.BlockSpec((NC, BR, EM), lambda i: (0, i, 0)),
          pl.BlockSpec((BR, EM), lambda i: (i, 0)),
          pl.BlockSpec((BR, 1), lambda i: (i, 0)),
          pl.BlockSpec((1, EM), lambda i: (0, 0)),
      ],
      out_specs=pl.BlockSpec((BR, EM), lambda i: (i, 0)),
      out_shape=jax.ShapeDtypeStruct((N, EM), jnp.float32),
  )(T2, y2, dis, b2.reshape(1, EM))

  return z


# speedup vs baseline: 1.1257x; 1.1257x over previous
import functools

import jax
import jax.numpy as jnp
from jax import lax
from jax.experimental import pallas as pl
from jax.experimental.pallas import tpu as pltpu
from jax.experimental.pallas import tpu_sc as plsc

NC = 2
NS = 16
NW = NC * NS
LANES = 16


def _mesh():
  return plsc.VectorSubcoreMesh(
      core_axis_name="c", subcore_axis_name="s", num_cores=NC, num_subcores=NS
  )


def _make_deg(E, NPAD):
  EPW = E // NW
  ROWS = NPAD // NS

  @functools.partial(
      pl.kernel,
      out_type=jax.ShapeDtypeStruct((NC, NPAD), jnp.float32),
      mesh=_mesh(),
      compiler_params=pltpu.CompilerParams(needs_layout_passes=False),
      scratch_types=[
          pltpu.VMEM((EPW,), jnp.int32),
          pltpu.VMEM((NPAD,), jnp.float32),
          pltpu.VMEM((ROWS,), jnp.float32),
          pltpu.VMEM((ROWS,), jnp.float32),
          pltpu.VMEM_SHARED((NS, NPAD), jnp.float32),
      ],
  )
  def deg_kernel(col_hbm, out_hbm, colv, hist, acc, tmp, shared):
    cid = lax.axis_index("c")
    sid = lax.axis_index("s")
    wid = sid * NC + cid
    pltpu.sync_copy(col_hbm.at[pl.ds(wid * EPW, EPW)], colv)

    zeros16 = jnp.zeros((LANES,), jnp.float32)

    def zero_body(i, _):
      hist[pl.ds(i * LANES, LANES)] = zeros16
      return ()

    lax.fori_loop(0, NPAD // LANES, zero_body, (), unroll=8)

    def hist_body(i, _):
      idx = colv[pl.ds(i * LANES, LANES)]
      cnt, last = plsc.scan_count(idx)
      plsc.addupdate_scatter(hist, [idx], cnt.astype(jnp.float32), mask=last)
      return ()

    lax.fori_loop(0, EPW // LANES, hist_body, (), unroll=4)

    pltpu.sync_copy(hist, shared.at[sid])
    plsc.subcore_barrier()

    base = sid * ROWS
    pltpu.sync_copy(shared.at[0, pl.ds(base, ROWS)], acc)

    def red_body(k, _):
      pltpu.sync_copy(shared.at[k, pl.ds(base, ROWS)], tmp)

      def add_body(i, _):
        s = pl.ds(i * LANES, LANES)
        acc[s] = acc[s] + tmp[s]
        return ()

      lax.fori_loop(0, ROWS // LANES, add_body, (), unroll=8)
      return ()

    lax.fori_loop(1, NS, red_body, ())
    pltpu.sync_copy(acc, out_hbm.at[cid, pl.ds(base, ROWS)])

  return deg_kernel


def _make_prop(E, NPAD, D, NCHUNK, CH):
  ROWS = NPAD // NS

  @functools.partial(
      pl.kernel,
      out_type=jax.ShapeDtypeStruct((NC, NPAD, D), jnp.float32),
      mesh=_mesh(),
      compiler_params=pltpu.CompilerParams(
          needs_layout_passes=False, use_tc_tiling_on_sc=False
      ),
      scratch_types=[
          pltpu.VMEM((NCHUNK, CH), jnp.int32),
          pltpu.VMEM((NCHUNK, CH), jnp.int32),
          pltpu.VMEM((16, CH, D), jnp.float32),
          pltpu.VMEM_SHARED((NPAD, D), jnp.float32),
          pltpu.SemaphoreType.DMA,
          pltpu.SemaphoreType.DMA,
      ],
  )
  def prop_kernel(rowr_hbm, colr_hbm, y_hbm, zeros_hbm, out_hbm,
                  rowv, colv, buf, acc, gsem, ssem):
    cid = lax.axis_index("c")
    sid = lax.axis_index("s")
    wid = sid * NC + cid
    base = sid * ROWS

    pltpu.sync_copy(zeros_hbm.at[pl.ds(base, ROWS)], acc.at[pl.ds(base, ROWS)])
    pltpu.sync_copy(rowr_hbm.at[wid], rowv)
    pltpu.sync_copy(colr_hbm.at[wid], colv)
    plsc.subcore_barrier()

    NBUF = 16
    for p in range(NBUF - 1):
      pltpu.async_copy(y_hbm.at[rowv.at[p]], buf.at[p], gsem)

    def step(j, _):
      slot = lax.rem(j, NBUF)
      pslot = lax.rem(j + NBUF - 1, NBUF)

      @pl.when(j >= 1)
      def _():
        pltpu.make_async_copy(buf.at[pslot], acc.at[colv.at[j - 1]],
                              ssem).wait()

      @pl.when(j + NBUF - 1 < NCHUNK)
      def _():
        pltpu.async_copy(y_hbm.at[rowv.at[j + NBUF - 1]], buf.at[pslot], gsem)

      pltpu.make_async_copy(y_hbm.at[rowv.at[j]], buf.at[slot], gsem).wait()
      pltpu.async_copy(buf.at[slot], acc.at[colv.at[j]], ssem, add=True)
      return ()

    lax.fori_loop(0, NCHUNK, step, ())
    last = NCHUNK - 1
    pltpu.make_async_copy(buf.at[lax.rem(last, NBUF)], acc.at[colv.at[last]],
                          ssem).wait()

    plsc.subcore_barrier()
    pltpu.sync_copy(acc.at[pl.ds(base, ROWS)],
                    out_hbm.at[cid, pl.ds(base, ROWS)])

  return prop_kernel


def _mm_scale_body(x_ref, w_ref, p_ref, dis_ref, y_ref):
  deg = p_ref[:, 0:1] + p_ref[:, 1:2] + 1.0
  dis = lax.rsqrt(deg)
  dis_ref[...] = dis
  xw = jnp.dot(x_ref[...], w_ref[...], preferred_element_type=jnp.float32)
  y_ref[...] = xw * dis


def _mid_body(t_ref, y1_ref, dis_ref, w2_ref, b1_ref, y2_ref):
  dis = dis_ref[...]
  t = jnp.sum(t_ref[...], axis=0) + y1_ref[...]
  h = jnp.maximum(t * dis + b1_ref[...], 0.0)
  y2_ref[...] = jnp.dot(h, w2_ref[...], preferred_element_type=jnp.float32) * dis


def _final_body(t_ref, y2_ref, dis_ref, b2_ref, z_ref):
  z_ref[...] = ((jnp.sum(t_ref[...], axis=0) + y2_ref[...]) * dis_ref[...]
                + b2_ref[...])


def kernel(x, ei, W1, b1, W2, b2):
  N, F = x.shape
  E = ei.shape[1]
  H = W1.shape[1]
  EM = W2.shape[1]

  assert E % (NW * 8) == 0
  EPW = E // NW
  CH = next(c for c in range(128, 0, -1) if EPW % c == 0)
  NCHUNK = EPW // CH
  ROWS = -(-N // NS)
  ROWS = -(ROWS // -LANES) * LANES
  NPAD = NS * ROWS

  BR = 400
  assert N % BR == 0
  GN = N // BR

  rowr = ei[0].reshape(NW, NCHUNK, CH)
  colr = ei[1].reshape(NW, NCHUNK, CH)
  zeros_h = jnp.zeros((NPAD, H), jnp.float32)
  zeros_e = jnp.zeros((NPAD, EM), jnp.float32)

  deg_fn = _make_deg(E, NPAD)
  prop_h = _make_prop(E, NPAD, H, NCHUNK, CH)
  prop_e = _make_prop(E, NPAD, EM, NCHUNK, CH)

  P = deg_fn(ei[1])
  Pt = P.T[:N]
  dis, y1 = pl.pallas_call(
      _mm_scale_body,
      grid=(GN,),
      in_specs=[
          pl.BlockSpec((BR, F), lambda i: (i, 0)),
          pl.BlockSpec((F, H), lambda i: (0, 0)),
          pl.BlockSpec((BR, NC), lambda i: (i, 0)),
      ],
      out_specs=[
          pl.BlockSpec((BR, 1), lambda i: (i, 0)),
          pl.BlockSpec((BR, H), lambda i: (i, 0)),
      ],
      out_shape=[
          jax.ShapeDtypeStruct((N, 1), jnp.float32),
          jax.ShapeDtypeStruct((N, H), jnp.float32),
      ],
  )(x, W1, Pt)

  T1 = prop_h(rowr, colr, y1, zeros_h)

  y2 = pl.pallas_call(
      _mid_body,
      grid=(GN,),
      in_specs=[
          pl.BlockSpec((NC, BR, H), lambda i: (0, i, 0)),
          pl.BlockSpec((BR, H), lambda i: (i, 0)),
          pl.BlockSpec((BR, 1), lambda i: (i, 0)),
          pl.BlockSpec((H, EM), lambda i: (0, 0)),
          pl.BlockSpec((1, H), lambda i: (0, 0)),
      ],
      out_specs=pl.BlockSpec((BR, EM), lambda i: (i, 0)),
      out_shape=jax.ShapeDtypeStruct((N, EM), jnp.float32),
  )(T1, y1, dis, W2, b1.reshape(1, H))

  T2 = prop_e(rowr, colr, y2, zeros_e)

  z = pl.pallas_call(
      _final_body,
      grid=(GN,),
      in_specs=[
          pl.BlockSpec((NC, BR, EM), lambda i: (0, i, 0)),
          pl.BlockSpec((BR, EM), lambda i: (i, 0)),
          pl.BlockSpec((BR, 1), lambda i: (i, 0)),
          pl.BlockSpec((1, EM), lambda i: (0, 0)),
      ],
      out_specs=pl.BlockSpec((BR, EM), lambda i: (i, 0)),
      out_shape=jax.ShapeDtypeStruct((N, EM), jnp.float32),
  )(T2, y2, dis, b2.reshape(1, EM))

  return z
